# baseline (speedup 1.0000x reference)
import jax
import jax.numpy as jnp
from jax import lax
from jax.experimental import pallas as pl

N_TOKENS = 32768
D_MODEL = 768
N_EXPERTS = 8
TEMPERATURE = 1.0

BLK = 8192


def _router_block(x_ref, w_ref, out_ref):
    xb = x_ref[...].astype(jnp.bfloat16)
    wb = w_ref[...].astype(jnp.bfloat16)
    out_ref[...] = lax.dot_general(
        xb, wb, (((1,), (1,)), ((), ())), preferred_element_type=jnp.float32
    )


def kernel(x, W):
    n_tokens, d_model = x.shape
    n_experts = W.shape[0]

    grid = (n_tokens // BLK,)
    logits = pl.pallas_call(
        _router_block,
        grid=grid,
        in_specs=[
            pl.BlockSpec((BLK, d_model), lambda i: (i, 0)),
            pl.BlockSpec((n_experts, d_model), lambda i: (0, 0)),
        ],
        out_specs=pl.BlockSpec((BLK, n_experts), lambda i: (i, 0)),
        out_shape=jax.ShapeDtypeStruct((n_tokens, n_experts), jnp.float32),
    )(x, W)

    temp = max(TEMPERATURE, 1e-06)
    if temp != 1.0:
        logits = logits / temp
    return logits


# speedup vs baseline: 1.0145x; 1.0145x over previous
import jax
import jax.numpy as jnp
from jax import lax
from jax.experimental import pallas as pl

N_TOKENS = 32768
D_MODEL = 768
N_EXPERTS = 8
TEMPERATURE = 1.0

BLK = 2048


def _router_block(x_ref, w_ref, out_ref):
    xb = x_ref[...].astype(jnp.bfloat16)
    wb = w_ref[...].astype(jnp.bfloat16)
    out_ref[...] = lax.dot_general(
        xb, wb, (((1,), (1,)), ((), ())), preferred_element_type=jnp.float32
    )


def kernel(x, W):
    n_tokens, d_model = x.shape
    n_experts = W.shape[0]

    grid = (n_tokens // BLK,)
    logits = pl.pallas_call(
        _router_block,
        grid=grid,
        in_specs=[
            pl.BlockSpec((BLK, d_model), lambda i: (i, 0)),
            pl.BlockSpec((n_experts, d_model), lambda i: (0, 0)),
        ],
        out_specs=pl.BlockSpec((BLK, n_experts), lambda i: (i, 0)),
        out_shape=jax.ShapeDtypeStruct((n_tokens, n_experts), jnp.float32),
    )(x, W)

    temp = max(TEMPERATURE, 1e-06)
    if temp != 1.0:
        logits = logits / temp
    return logits


# speedup vs baseline: 1.0363x; 1.0214x over previous
import jax
import jax.numpy as jnp
from jax import lax
from jax.experimental import pallas as pl

N_TOKENS = 32768
D_MODEL = 768
N_EXPERTS = 8
TEMPERATURE = 1.0

BLK = 4096


def _router_block(x_ref, w_ref, out_ref):
    xb = x_ref[...].astype(jnp.bfloat16)
    wb = w_ref[...].astype(jnp.bfloat16)
    out_ref[...] = lax.dot_general(
        xb, wb, (((1,), (1,)), ((), ())), preferred_element_type=jnp.float32
    )


def kernel(x, W):
    n_tokens, d_model = x.shape
    n_experts = W.shape[0]

    grid = (n_tokens // BLK,)
    logits = pl.pallas_call(
        _router_block,
        grid=grid,
        in_specs=[
            pl.BlockSpec((BLK, d_model), lambda i: (i, 0)),
            pl.BlockSpec((n_experts, d_model), lambda i: (0, 0)),
        ],
        out_specs=pl.BlockSpec((BLK, n_experts), lambda i: (i, 0)),
        out_shape=jax.ShapeDtypeStruct((n_tokens, n_experts), jnp.float32),
    )(x, W)

    temp = max(TEMPERATURE, 1e-06)
    if temp != 1.0:
        logits = logits / temp
    return logits
